# baseline (speedup 1.0000x reference)
import jax
import jax.numpy as jnp
from jax.experimental import pallas as pl

HIDDEN = 128
NUM_LAYERS = 7
MSG_EPS = 1e-7


def _layer_norm(h, g, b):
    mu = jnp.mean(h, axis=-1, keepdims=True)
    var = jnp.var(h, axis=-1, keepdims=True)
    return g * (h - mu) / jnp.sqrt(var + 1e-5) + b


def _scatter_softmax(msg, dst, n):
    mmax = jax.ops.segment_max(msg, dst, num_segments=n)
    mmax = jnp.where(jnp.isfinite(mmax), mmax, 0.0)
    ex = jnp.exp(msg - mmax[dst])
    den = jax.ops.segment_sum(ex, dst, num_segments=n)
    return ex / (den[dst] + 1e-16)


def _gen_conv(h, src, dst, edge_emb, W, b, t, n):
    msg = jax.nn.relu(h[src] + edge_emb) + MSG_EPS
    alpha = _scatter_softmax(msg * t, dst, n)
    m = jax.ops.segment_sum(msg * alpha, dst, num_segments=n)
    out = h + m
    return out @ W + b


def _proj_kernel(h_ref, w_ref, b_ref, o_ref):
    o_ref[...] = jnp.dot(h_ref[...], w_ref[...],
                         preferred_element_type=jnp.float32) + b_ref[...]


def _final_proj(h, W_pred, b_pred):
    n, k = h.shape
    m = W_pred.shape[1]
    return pl.pallas_call(
        _proj_kernel,
        out_shape=jax.ShapeDtypeStruct((n, m), jnp.float32),
    )(h, W_pred, b_pred.reshape(1, m))


def kernel(x, node_index, edge_index, edge_attr, node_features, W_nf, b_nf,
           W_edge, b_edge, Wg, bg, ln_g, ln_b, t, W_pred, b_pred):
    n = node_index.shape[0]
    src = edge_index[0]
    dst = edge_index[1]
    nf = node_features[node_index]
    h = nf @ W_nf + b_nf
    edge_emb = edge_attr @ W_edge + b_edge
    h = _gen_conv(h, src, dst, edge_emb, Wg[0], bg[0], t[0], n)
    for layer in range(1, NUM_LAYERS):
        h1 = _layer_norm(h, ln_g[layer - 1], ln_b[layer - 1])
        h2 = jax.nn.relu(h1)
        h = _gen_conv(h2, src, dst, edge_emb, Wg[layer], bg[layer], t[layer], n) + h
    h = jax.nn.relu(_layer_norm(h, ln_g[NUM_LAYERS - 1], ln_b[NUM_LAYERS - 1]))
    return _final_proj(h, W_pred, b_pred)


# speedup vs baseline: 1.9918x; 1.9918x over previous
import functools

import jax
import jax.numpy as jnp
from jax import lax
from jax.experimental import pallas as pl
from jax.experimental.pallas import tpu as pltpu
from jax.experimental.pallas import tpu_sc as plsc

N = 10000
NP = 10240
E = 320000
H = 128
HH = 64
NUM_LAYERS = 7
NUM_TASKS = 112
MSG_EPS = 1e-7

NC, NS, L = 2, 16, 16
CH = 64
EPT = 20096
E_PAD = NS * EPT
NCHUNK = EPT // CH
ACC_ROWS = NP
DEAD_ROW = N + 100
NPT = NP // NS
GR = NP // (NC * NS)
GC = 64

_MESH = plsc.VectorSubcoreMesh(core_axis_name="c", subcore_axis_name="s")


@functools.partial(
    pl.kernel,
    mesh=_MESH,
    out_type=jax.ShapeDtypeStruct((2 * NP, HH), jnp.float32),
    scratch_types=[
        pltpu.VMEM_SHARED((ACC_ROWS, H), jnp.float32),
        pltpu.VMEM((CH,), jnp.int32),
        pltpu.VMEM((CH,), jnp.int32),
        pltpu.VMEM((CH, H), jnp.float32),
        pltpu.VMEM((CH, HH), jnp.float32),
        pltpu.VMEM((CH, H), jnp.float32),
        pltpu.VMEM((L,), jnp.float32),
        pltpu.SemaphoreType.DMA,
    ],
)
def _sc_edge(h2s, srcp, dstp, eep, tvec, m_out,
             acc, s_idx, d_idx, rows, eebuf, pq, tvm, sem):
    c = lax.axis_index("c")
    s = lax.axis_index("s")
    co = c * HH

    pltpu.sync_copy(tvec, tvm)
    tv = tvm[...]

    def zrow(r, carry):
        for k in range(8):
            pq[r, pl.ds(k * 16, 16)] = jnp.zeros((16,), jnp.float32)
        return carry

    lax.fori_loop(0, CH, zrow, 0)
    for g in range(ACC_ROWS // NS // CH):
        pltpu.sync_copy(pq, acc.at[pl.ds(s * (ACC_ROWS // NS) + g * CH, CH)])
    plsc.subcore_barrier()

    ebase = s * EPT

    def chunk(g, carry):
        base = ebase + g * CH
        pltpu.sync_copy(srcp.at[pl.ds(base, CH)], s_idx)
        pltpu.sync_copy(dstp.at[pl.ds(base, CH)], d_idx)
        pltpu.async_copy(h2s.at[s_idx], rows, sem).wait()
        pltpu.sync_copy(eep.at[pl.ds(c * E_PAD + base, CH)], eebuf)

        def row(r, rc):
            for k in range(4):
                sl = pl.ds(k * 16, 16)
                msg = (jnp.maximum(rows[r, pl.ds(co + k * 16, 16)]
                                   + eebuf[r, sl], 0.0) + MSG_EPS)
                p = jnp.exp(msg * tv)
                pq[r, pl.ds(HH + k * 16, 16)] = p
                pq[r, sl] = msg * p
            return rc

        lax.fori_loop(0, CH, row, 0)
        pltpu.sync_copy(pq, acc.at[d_idx], add=True)
        return carry

    lax.fori_loop(0, NCHUNK, chunk, 0)
    plsc.subcore_barrier()

    for j in range(NPT // CH):
        r0 = s * NPT + j * CH
        pltpu.sync_copy(acc.at[pl.ds(r0, CH)], rows)

        def mrow(r, rc):
            for k in range(4):
                sl = pl.ds(k * 16, 16)
                q = rows[r, sl]
                p = rows[r, pl.ds(HH + k * 16, 16)]
                eebuf[r, sl] = q / (p + 1e-16)
            return rc

        lax.fori_loop(0, CH, mrow, 0)
        pltpu.sync_copy(eebuf, m_out.at[pl.ds(c * NP + r0, CH)])


@functools.partial(
    pl.kernel,
    mesh=_MESH,
    out_type=jax.ShapeDtypeStruct((NP, H), jnp.float32),
    scratch_types=[
        pltpu.VMEM((GC,), jnp.int32),
        pltpu.VMEM((GR, H), jnp.float32),
        pltpu.SemaphoreType.DMA,
    ],
)
def _sc_gather(h0src, nidx, h0s, g_idx, gbuf, sem):
    c = lax.axis_index("c")
    s = lax.axis_index("s")
    rbase = (c * NS + s) * GR
    for g in range(GR // GC):
        pltpu.sync_copy(nidx.at[pl.ds(rbase + g * GC, GC)], g_idx)
        pltpu.async_copy(h0src.at[g_idx], gbuf.at[pl.ds(g * GC, GC)], sem).wait()
    pltpu.sync_copy(gbuf, h0s.at[pl.ds(rbase, GR)])


def _tc_in_proj_half_body(a_ref, w_ref, b_ref, o_ref):
    y = jnp.dot(a_ref[...], w_ref[...],
                preferred_element_type=jnp.float32) + b_ref[...]
    o_ref[0] = y[:, :HH]
    o_ref[1] = y[:, HH:]


def _in_proj_half(a, w, b, br):
    n = a.shape[0]
    return pl.pallas_call(
        _tc_in_proj_half_body,
        grid=(n // br,),
        in_specs=[
            pl.BlockSpec((br, 8), lambda i: (i, 0)),
            pl.BlockSpec((8, H), lambda i: (0, 0)),
            pl.BlockSpec((1, H), lambda i: (0, 0)),
        ],
        out_specs=pl.BlockSpec((2, br, HH), lambda i: (0, i, 0)),
        out_shape=jax.ShapeDtypeStruct((2, n, HH), jnp.float32),
    )(a, w, b.reshape(1, H))


def _tc_in_proj_full_body(a_ref, w_ref, b_ref, o_ref):
    o_ref[...] = jnp.dot(a_ref[...], w_ref[...],
                         preferred_element_type=jnp.float32) + b_ref[...]


def _in_proj_full(a, w, b, br):
    n = a.shape[0]
    return pl.pallas_call(
        _tc_in_proj_full_body,
        grid=(n // br,),
        in_specs=[
            pl.BlockSpec((br, 8), lambda i: (i, 0)),
            pl.BlockSpec((8, H), lambda i: (0, 0)),
            pl.BlockSpec((1, H), lambda i: (0, 0)),
        ],
        out_specs=pl.BlockSpec((br, H), lambda i: (i, 0)),
        out_shape=jax.ShapeDtypeStruct((n, H), jnp.float32),
    )(a, w, b.reshape(1, H))


def _ln_relu(hn, g, b):
    mu = jnp.mean(hn, axis=-1, keepdims=True)
    var = jnp.mean((hn - mu) ** 2, axis=-1, keepdims=True)
    return jnp.maximum(g * (hn - mu) / jnp.sqrt(var + 1e-5) + b, 0.0)


def _tc_layer_res_body(m_ref, h2_ref, hp_ref, w_ref, b_ref, g_ref, bb_ref,
                       hn_ref, h2n_ref):
    m = jnp.concatenate([m_ref[0], m_ref[1]], axis=-1)
    u = jnp.dot(h2_ref[...] + m, w_ref[...],
                preferred_element_type=jnp.float32) + b_ref[...]
    hn = u + hp_ref[...]
    hn_ref[...] = hn
    h2n_ref[...] = _ln_relu(hn, g_ref[...], bb_ref[...])


def _tc_layer0_body(m_ref, h2_ref, w_ref, b_ref, g_ref, bb_ref,
                    hn_ref, h2n_ref):
    m = jnp.concatenate([m_ref[0], m_ref[1]], axis=-1)
    hn = jnp.dot(h2_ref[...] + m, w_ref[...],
                 preferred_element_type=jnp.float32) + b_ref[...]
    hn_ref[...] = hn
    h2n_ref[...] = _ln_relu(hn, g_ref[...], bb_ref[...])


def _tc_layer(m2, h2, hp, w, b, g, bb, br=1024, residual=True):
    half = pl.BlockSpec((2, br, HH), lambda i: (0, i, 0))
    full = pl.BlockSpec((br, H), lambda i: (i, 0))
    wspec = pl.BlockSpec((H, H), lambda i: (0, 0))
    vspec = pl.BlockSpec((1, H), lambda i: (0, 0))
    body = _tc_layer_res_body if residual else _tc_layer0_body
    in_specs = [half, full] + ([full] if residual else []) + [wspec, vspec, vspec, vspec]
    args = [m2, h2] + ([hp] if residual else []) + \
        [w, b.reshape(1, H), g.reshape(1, H), bb.reshape(1, H)]
    return pl.pallas_call(
        body,
        grid=(NP // br,),
        in_specs=in_specs,
        out_specs=[full, full],
        out_shape=[jax.ShapeDtypeStruct((NP, H), jnp.float32),
                   jax.ShapeDtypeStruct((NP, H), jnp.float32)],
    )(*args)


def _tc_final_body(m_ref, h2_ref, hp_ref, w_ref, b_ref, g_ref, bb_ref,
                   wp_ref, bp_ref, o_ref):
    m = jnp.concatenate([m_ref[0], m_ref[1]], axis=-1)
    u = jnp.dot(h2_ref[...] + m, w_ref[...],
                preferred_element_type=jnp.float32) + b_ref[...]
    hn = u + hp_ref[...]
    y = _ln_relu(hn, g_ref[...], bb_ref[...])
    o_ref[...] = jnp.dot(y, wp_ref[...],
                         preferred_element_type=jnp.float32) + bp_ref[...]


def _tc_final(m2, h2, hp, w, b, g, bb, wp, bp, br=1024):
    half = pl.BlockSpec((2, br, HH), lambda i: (0, i, 0))
    full = pl.BlockSpec((br, H), lambda i: (i, 0))
    wspec = pl.BlockSpec((H, H), lambda i: (0, 0))
    vspec = pl.BlockSpec((1, H), lambda i: (0, 0))
    return pl.pallas_call(
        _tc_final_body,
        grid=(NP // br,),
        in_specs=[half, full, full, wspec, vspec, vspec, vspec,
                  pl.BlockSpec((H, NUM_TASKS), lambda i: (0, 0)),
                  pl.BlockSpec((1, NUM_TASKS), lambda i: (0, 0))],
        out_specs=pl.BlockSpec((br, NUM_TASKS), lambda i: (i, 0)),
        out_shape=jax.ShapeDtypeStruct((NP, NUM_TASKS), jnp.float32),
    )(m2, h2, hp, w, b.reshape(1, H), g.reshape(1, H), bb.reshape(1, H),
      wp, bp.reshape(1, NUM_TASKS))


def kernel(x, node_index, edge_index, edge_attr, node_features, W_nf, b_nf,
           W_edge, b_edge, Wg, bg, ln_g, ln_b, t, W_pred, b_pred):
    src = edge_index[0].astype(jnp.int32)
    dst = edge_index[1].astype(jnp.int32)
    nodei = node_index.astype(jnp.int32)

    srcp = jnp.zeros((E_PAD,), jnp.int32).at[:E].set(src)
    dstp = jnp.full((E_PAD,), DEAD_ROW, jnp.int32).at[:E].set(dst)
    eap = jnp.zeros((E_PAD, 8), jnp.float32).at[:E].set(edge_attr)
    nip = jnp.zeros((NP,), jnp.int32).at[:N].set(nodei)
    tvs = jnp.broadcast_to(t.reshape(NUM_LAYERS, 1), (NUM_LAYERS, L))
    nfp = jnp.zeros((NP, 8), jnp.float32).at[:N].set(node_features)

    h0f = _in_proj_full(nfp, W_nf, b_nf, br=1024)
    eeh = _in_proj_half(eap, W_edge, b_edge, br=2048)
    eep = eeh.reshape(2 * E_PAD, HH)

    h2 = _sc_gather(h0f, nip)
    h = None
    for l in range(NUM_LAYERS):
        m_f = _sc_edge(h2, srcp, dstp, eep, tvs[l])
        m2 = m_f.reshape(2, NP, HH)
        if l == 0:
            h, h2 = _tc_layer(m2, h2, None, Wg[0], bg[0], ln_g[0], ln_b[0],
                              residual=False)
        elif l < NUM_LAYERS - 1:
            h, h2 = _tc_layer(m2, h2, h, Wg[l], bg[l], ln_g[l], ln_b[l])
        else:
            out = _tc_final(m2, h2, h, Wg[l], bg[l],
                            ln_g[l], ln_b[l], W_pred, b_pred)
            return out[:N]


# speedup vs baseline: 2.7169x; 1.3640x over previous
import functools

import jax
import jax.numpy as jnp
from jax import lax
from jax.experimental import pallas as pl
from jax.experimental.pallas import tpu as pltpu
from jax.experimental.pallas import tpu_sc as plsc

N = 10000
NP = 10240
E = 320000
H = 128
HH = 64
NUM_LAYERS = 7
NUM_TASKS = 112
MSG_EPS = 1e-7

NC, NS, L = 2, 16, 16
C = 32
CPS = 64
SUP = C * CPS
NSUPER = 10
EPT = SUP * NSUPER
E_PAD = NS * EPT
ACC_ROWS = NP
DEAD_ROW = N + 100
NPT = NP // NS
GR = NP // (NC * NS)
GC = 64

_MESH = plsc.VectorSubcoreMesh(core_axis_name="c", subcore_axis_name="s")


@functools.partial(
    pl.kernel,
    mesh=_MESH,
    out_type=jax.ShapeDtypeStruct((2 * NP, HH), jnp.float32),
    compiler_params=pltpu.CompilerParams(use_tc_tiling_on_sc=False),
    scratch_types=[
        pltpu.VMEM_SHARED((ACC_ROWS, H), jnp.float32),
        pltpu.VMEM((CPS, C), jnp.int32),
        pltpu.VMEM((CPS, C), jnp.int32),
        pltpu.VMEM((CPS, C), jnp.int32),
        pltpu.VMEM((CPS, C), jnp.int32),
        pltpu.VMEM((C, H), jnp.float32),
        pltpu.VMEM((C, H), jnp.float32),
        pltpu.VMEM((C, HH), jnp.float32),
        pltpu.VMEM((C, HH), jnp.float32),
        pltpu.VMEM((C, H), jnp.float32),
        pltpu.VMEM((L,), jnp.float32),
    ] + [pltpu.SemaphoreType.DMA] * 6,
)
def _sc_edge(h2s, src2d, dst2d, eep, tvec, m_out, acc,
             si0, si1, di0, di1, rows0, rows1, ee0, ee1, pq0, tvm,
             is0, is1, gs0, gs1, es0, es1):
    c = lax.axis_index("c")
    s = lax.axis_index("s")
    co = c * HH
    SI, DI = (si0, si1), (di0, di1)
    RO, EB = (rows0, rows1), (ee0, ee1)
    IS, GS, ES = (is0, is1), (gs0, gs1), (es0, es1)

    pltpu.sync_copy(tvec, tvm)
    tv = tvm[...]

    def zrow(r, carry):
        for k in range(8):
            pq0[r, pl.ds(k * 16, 16)] = jnp.zeros((16,), jnp.float32)
        return carry

    lax.fori_loop(0, C, zrow, 0)
    for g in range(NPT // C):
        pltpu.sync_copy(pq0, acc.at[pl.ds(s * NPT + g * C, C)])
    plsc.subcore_barrier()

    irow = s * (EPT // C)
    ebase = s * EPT

    def issue_gather(sb, j, b):
        pltpu.async_copy(h2s.at[SI[sb].at[j]], RO[b], GS[b])

    def issue_ee(S, j, b):
        off = c * E_PAD + ebase + S * SUP + j * C
        pltpu.async_copy(eep.at[pl.ds(off, C)], EB[b], ES[b])

    def drain(dummy, buf, sem):
        pltpu.make_async_copy(dummy, buf, sem).wait()

    def compute(j, b, sb):
        drain(h2s.at[pl.ds(0, C)], RO[b], GS[b])
        drain(eep.at[pl.ds(0, C)], EB[b], ES[b])

        def rowf(r, rc):
            for k in range(4):
                sl = pl.ds(k * 16, 16)
                msg = (jnp.maximum(RO[b][r, pl.ds(co + k * 16, 16)]
                                   + EB[b][r, sl], 0.0) + MSG_EPS)
                p = jnp.exp(msg * tv)
                pq0[r, pl.ds(HH + k * 16, 16)] = p
                pq0[r, sl] = msg * p
            return rc

        lax.fori_loop(0, C, rowf, 0)
        pltpu.sync_copy(pq0, acc.at[DI[sb].at[j]], add=True)

    pltpu.sync_copy(src2d.at[pl.ds(irow, CPS)], si0)
    pltpu.sync_copy(dst2d.at[pl.ds(irow, CPS)], di0)
    for b in range(2):
        issue_gather(0, b, b)
        issue_ee(0, b, b)

    for S in range(NSUPER):
        sb = S % 2
        nb = 1 - sb
        if S + 1 < NSUPER:
            pltpu.async_copy(src2d.at[pl.ds(irow + (S + 1) * CPS, CPS)],
                             SI[nb], IS[nb])
            pltpu.async_copy(dst2d.at[pl.ds(irow + (S + 1) * CPS, CPS)],
                             DI[nb], IS[nb])
        if S == 0:
            for b in range(2):
                compute(b, b, sb)
                issue_gather(sb, b + 2, b)
                issue_ee(S, b + 2, b)
            p_lo = 1
        else:
            p_lo = 0

        def pairf(p, carry, _S=S, _sb=sb):
            for b in range(2):
                j = 2 * p + b
                compute(j, b, _sb)

                @pl.when(p < CPS // 2 - 1)
                def _():
                    issue_gather(_sb, j + 2, b)
                    issue_ee(_S, j + 2, b)
            return carry

        lax.fori_loop(p_lo, CPS // 2, pairf, 0)

        if S + 1 < NSUPER:
            drain(src2d.at[pl.ds(0, CPS)], SI[nb], IS[nb])
            drain(src2d.at[pl.ds(0, CPS)], DI[nb], IS[nb])
            for b in range(2):
                issue_gather(nb, b, b)
                issue_ee(S + 1, b, b)

    plsc.subcore_barrier()

    for g in range(NPT // C):
        r0 = s * NPT + g * C
        pltpu.sync_copy(acc.at[pl.ds(r0, C)], rows0)

        def mrow(r, rc):
            for k in range(4):
                sl = pl.ds(k * 16, 16)
                q = rows0[r, sl]
                p = rows0[r, pl.ds(HH + k * 16, 16)]
                ee0[r, sl] = q / (p + 1e-16)
            return rc

        lax.fori_loop(0, C, mrow, 0)
        pltpu.sync_copy(ee0, m_out.at[pl.ds(c * NP + r0, C)])


@functools.partial(
    pl.kernel,
    mesh=_MESH,
    out_type=jax.ShapeDtypeStruct((NP, H), jnp.float32),
    compiler_params=pltpu.CompilerParams(use_tc_tiling_on_sc=False),
    scratch_types=[
        pltpu.VMEM((GC,), jnp.int32),
        pltpu.VMEM((GR, H), jnp.float32),
        pltpu.SemaphoreType.DMA,
    ],
)
def _sc_gather(h0src, nidx, h0s, g_idx, gbuf, sem):
    c = lax.axis_index("c")
    s = lax.axis_index("s")
    rbase = (c * NS + s) * GR
    for g in range(GR // GC):
        pltpu.sync_copy(nidx.at[pl.ds(rbase + g * GC, GC)], g_idx)
        pltpu.async_copy(h0src.at[g_idx], gbuf.at[pl.ds(g * GC, GC)], sem).wait()
    pltpu.sync_copy(gbuf, h0s.at[pl.ds(rbase, GR)])


def _tc_in_proj_half_body(a_ref, w_ref, b_ref, o_ref):
    y = jnp.dot(a_ref[...], w_ref[...],
                preferred_element_type=jnp.float32) + b_ref[...]
    o_ref[0] = y[:, :HH]
    o_ref[1] = y[:, HH:]


def _in_proj_half(a, w, b, br):
    n = a.shape[0]
    return pl.pallas_call(
        _tc_in_proj_half_body,
        grid=(n // br,),
        in_specs=[
            pl.BlockSpec((br, 8), lambda i: (i, 0)),
            pl.BlockSpec((8, H), lambda i: (0, 0)),
            pl.BlockSpec((1, H), lambda i: (0, 0)),
        ],
        out_specs=pl.BlockSpec((2, br, HH), lambda i: (0, i, 0)),
        out_shape=jax.ShapeDtypeStruct((2, n, HH), jnp.float32),
    )(a, w, b.reshape(1, H))


def _tc_in_proj_full_body(a_ref, w_ref, b_ref, o_ref):
    o_ref[...] = jnp.dot(a_ref[...], w_ref[...],
                         preferred_element_type=jnp.float32) + b_ref[...]


def _in_proj_full(a, w, b, br):
    n = a.shape[0]
    return pl.pallas_call(
        _tc_in_proj_full_body,
        grid=(n // br,),
        in_specs=[
            pl.BlockSpec((br, 8), lambda i: (i, 0)),
            pl.BlockSpec((8, H), lambda i: (0, 0)),
            pl.BlockSpec((1, H), lambda i: (0, 0)),
        ],
        out_specs=pl.BlockSpec((br, H), lambda i: (i, 0)),
        out_shape=jax.ShapeDtypeStruct((n, H), jnp.float32),
    )(a, w, b.reshape(1, H))


def _ln_relu(hn, g, b):
    mu = jnp.mean(hn, axis=-1, keepdims=True)
    var = jnp.mean((hn - mu) ** 2, axis=-1, keepdims=True)
    return jnp.maximum(g * (hn - mu) / jnp.sqrt(var + 1e-5) + b, 0.0)


def _tc_layer_res_body(m_ref, h2_ref, hp_ref, w_ref, b_ref, g_ref, bb_ref,
                       hn_ref, h2n_ref):
    m = jnp.concatenate([m_ref[0], m_ref[1]], axis=-1)
    u = jnp.dot(h2_ref[...] + m, w_ref[...],
                preferred_element_type=jnp.float32) + b_ref[...]
    hn = u + hp_ref[...]
    hn_ref[...] = hn
    h2n_ref[...] = _ln_relu(hn, g_ref[...], bb_ref[...])


def _tc_layer0_body(m_ref, h2_ref, w_ref, b_ref, g_ref, bb_ref,
                    hn_ref, h2n_ref):
    m = jnp.concatenate([m_ref[0], m_ref[1]], axis=-1)
    hn = jnp.dot(h2_ref[...] + m, w_ref[...],
                 preferred_element_type=jnp.float32) + b_ref[...]
    hn_ref[...] = hn
    h2n_ref[...] = _ln_relu(hn, g_ref[...], bb_ref[...])


def _tc_layer(m2, h2, hp, w, b, g, bb, br=1024, residual=True):
    half = pl.BlockSpec((2, br, HH), lambda i: (0, i, 0))
    full = pl.BlockSpec((br, H), lambda i: (i, 0))
    wspec = pl.BlockSpec((H, H), lambda i: (0, 0))
    vspec = pl.BlockSpec((1, H), lambda i: (0, 0))
    body = _tc_layer_res_body if residual else _tc_layer0_body
    in_specs = [half, full] + ([full] if residual else []) + [wspec, vspec, vspec, vspec]
    args = [m2, h2] + ([hp] if residual else []) + \
        [w, b.reshape(1, H), g.reshape(1, H), bb.reshape(1, H)]
    return pl.pallas_call(
        body,
        grid=(NP // br,),
        in_specs=in_specs,
        out_specs=[full, full],
        out_shape=[jax.ShapeDtypeStruct((NP, H), jnp.float32),
                   jax.ShapeDtypeStruct((NP, H), jnp.float32)],
    )(*args)


def _tc_final_body(m_ref, h2_ref, hp_ref, w_ref, b_ref, g_ref, bb_ref,
                   wp_ref, bp_ref, o_ref):
    m = jnp.concatenate([m_ref[0], m_ref[1]], axis=-1)
    u = jnp.dot(h2_ref[...] + m, w_ref[...],
                preferred_element_type=jnp.float32) + b_ref[...]
    hn = u + hp_ref[...]
    y = _ln_relu(hn, g_ref[...], bb_ref[...])
    o_ref[...] = jnp.dot(y, wp_ref[...],
                         preferred_element_type=jnp.float32) + bp_ref[...]


def _tc_final(m2, h2, hp, w, b, g, bb, wp, bp, br=1024):
    half = pl.BlockSpec((2, br, HH), lambda i: (0, i, 0))
    full = pl.BlockSpec((br, H), lambda i: (i, 0))
    wspec = pl.BlockSpec((H, H), lambda i: (0, 0))
    vspec = pl.BlockSpec((1, H), lambda i: (0, 0))
    return pl.pallas_call(
        _tc_final_body,
        grid=(NP // br,),
        in_specs=[half, full, full, wspec, vspec, vspec, vspec,
                  pl.BlockSpec((H, NUM_TASKS), lambda i: (0, 0)),
                  pl.BlockSpec((1, NUM_TASKS), lambda i: (0, 0))],
        out_specs=pl.BlockSpec((br, NUM_TASKS), lambda i: (i, 0)),
        out_shape=jax.ShapeDtypeStruct((NP, NUM_TASKS), jnp.float32),
    )(m2, h2, hp, w, b.reshape(1, H), g.reshape(1, H), bb.reshape(1, H),
      wp, bp.reshape(1, NUM_TASKS))


def kernel(x, node_index, edge_index, edge_attr, node_features, W_nf, b_nf,
           W_edge, b_edge, Wg, bg, ln_g, ln_b, t, W_pred, b_pred):
    src = edge_index[0].astype(jnp.int32)
    dst = edge_index[1].astype(jnp.int32)
    nodei = node_index.astype(jnp.int32)

    srcp = jnp.zeros((E_PAD,), jnp.int32).at[:E].set(src)
    dstp = jnp.full((E_PAD,), DEAD_ROW, jnp.int32).at[:E].set(dst)
    eap = jnp.zeros((E_PAD, 8), jnp.float32).at[:E].set(edge_attr)
    nip = jnp.zeros((NP,), jnp.int32).at[:N].set(nodei)
    tvs = jnp.broadcast_to(t.reshape(NUM_LAYERS, 1), (NUM_LAYERS, L))
    nfp = jnp.zeros((NP, 8), jnp.float32).at[:N].set(node_features)

    h0f = _in_proj_full(nfp, W_nf, b_nf, br=1024)
    eeh = _in_proj_half(eap, W_edge, b_edge, br=2048)
    eep = eeh.reshape(2 * E_PAD, HH)

    src2d = srcp.reshape(E_PAD // C, C)
    dst2d = dstp.reshape(E_PAD // C, C)

    h2 = _sc_gather(h0f, nip)
    h = None
    for l in range(NUM_LAYERS):
        m_f = _sc_edge(h2, src2d, dst2d, eep, tvs[l])
        m2 = m_f.reshape(2, NP, HH)
        if l == 0:
            h, h2 = _tc_layer(m2, h2, None, Wg[0], bg[0], ln_g[0], ln_b[0],
                              residual=False)
        elif l < NUM_LAYERS - 1:
            h, h2 = _tc_layer(m2, h2, h, Wg[l], bg[l], ln_g[l], ln_b[l])
        else:
            out = _tc_final(m2, h2, h, Wg[l], bg[l],
                            ln_g[l], ln_b[l], W_pred, b_pred)
            return out[:N]


# speedup vs baseline: 4.4984x; 1.6557x over previous
import functools

import jax
import jax.numpy as jnp
from jax import lax
from jax.experimental import pallas as pl
from jax.experimental.pallas import tpu as pltpu
from jax.experimental.pallas import tpu_sc as plsc

N = 10000
NP = 10240
E = 320000
H = 128
HH = 64
NUM_LAYERS = 7
NUM_TASKS = 112
MSG_EPS = 1e-7

NC, NS, L = 2, 16, 16
C = 32
CPS = 64
SUP = C * CPS
NSUPER = 10
EPT = SUP * NSUPER
E_PAD = NS * EPT
ACC_ROWS = NP
DEAD_ROW = N + 100
NPT = NP // NS
GR = NP // (NC * NS)
GC = 64

_MESH = plsc.VectorSubcoreMesh(core_axis_name="c", subcore_axis_name="s")


@functools.partial(
    pl.kernel,
    mesh=_MESH,
    out_type=jax.ShapeDtypeStruct((2 * NP, HH), jnp.float32),
    compiler_params=pltpu.CompilerParams(use_tc_tiling_on_sc=False),
    scratch_types=[
        pltpu.VMEM_SHARED((ACC_ROWS, H), jnp.float32),
        pltpu.VMEM((CPS, C), jnp.int32),
        pltpu.VMEM((CPS, C), jnp.int32),
        pltpu.VMEM((CPS, C), jnp.int32),
        pltpu.VMEM((CPS, C), jnp.int32),
        pltpu.VMEM((C, H), jnp.float32),
        pltpu.VMEM((C, H), jnp.float32),
        pltpu.VMEM((C, HH), jnp.float32),
        pltpu.VMEM((C, HH), jnp.float32),
        pltpu.VMEM((C, H), jnp.float32),
        pltpu.VMEM((L,), jnp.float32),
    ] + [pltpu.SemaphoreType.DMA] * 6,
)
def _sc_edge(h2s, src2d, dst2d, eep, tvec, m_out, acc,
             si0, si1, di0, di1, rows0, rows1, ee0, ee1, pq0, tvm,
             is0, is1, gs0, gs1, es0, es1):
    c = lax.axis_index("c")
    s = lax.axis_index("s")
    co = c * HH
    SI, DI = (si0, si1), (di0, di1)
    RO, EB = (rows0, rows1), (ee0, ee1)
    IS, GS, ES = (is0, is1), (gs0, gs1), (es0, es1)

    pltpu.sync_copy(tvec, tvm)
    tv = tvm[...]

    def zrow(r, carry):
        for k in range(8):
            pq0[r, pl.ds(k * 16, 16)] = jnp.zeros((16,), jnp.float32)
        return carry

    lax.fori_loop(0, C, zrow, 0)

    def zcp(g, carry):
        pltpu.sync_copy(pq0, acc.at[pl.ds(s * NPT + g * C, C)])
        return carry

    lax.fori_loop(0, NPT // C, zcp, 0)
    plsc.subcore_barrier()

    irow = s * (EPT // C)
    ebase = s * EPT

    def issue_gather(sb, j, b):
        pltpu.async_copy(h2s.at[SI[sb].at[j]], RO[b], GS[b])

    def issue_ee(S, j, b):
        off = c * E_PAD + ebase + S * SUP + j * C
        pltpu.async_copy(eep.at[pl.ds(off, C)], EB[b], ES[b])

    def drain(dummy, buf, sem):
        pltpu.make_async_copy(dummy, buf, sem).wait()

    def compute(j, b, sb):
        drain(h2s.at[pl.ds(0, C)], RO[b], GS[b])
        drain(eep.at[pl.ds(0, C)], EB[b], ES[b])

        @plsc.parallel_loop(0, C, unroll=2)
        def _(r):
            for k in range(4):
                sl = pl.ds(k * 16, 16)
                msg = (jnp.maximum(RO[b][r, pl.ds(co + k * 16, 16)]
                                   + EB[b][r, sl], 0.0) + MSG_EPS)
                p = jnp.exp(msg * tv)
                pq0[r, pl.ds(HH + k * 16, 16)] = p
                pq0[r, sl] = msg * p

        pltpu.sync_copy(pq0, acc.at[DI[sb].at[j]], add=True)

    pltpu.sync_copy(src2d.at[pl.ds(irow, CPS)], si0)
    pltpu.sync_copy(dst2d.at[pl.ds(irow, CPS)], di0)
    for b in range(2):
        issue_gather(0, b, b)
        issue_ee(0, b, b)

    for S in range(NSUPER):
        sb = S % 2
        nb = 1 - sb
        if S + 1 < NSUPER:
            pltpu.async_copy(src2d.at[pl.ds(irow + (S + 1) * CPS, CPS)],
                             SI[nb], IS[nb])
            pltpu.async_copy(dst2d.at[pl.ds(irow + (S + 1) * CPS, CPS)],
                             DI[nb], IS[nb])
        if S == 0:
            for b in range(2):
                compute(b, b, sb)
                issue_gather(sb, b + 2, b)
                issue_ee(S, b + 2, b)
            p_lo = 1
        else:
            p_lo = 0

        def pairf(p, carry, _S=S, _sb=sb):
            for b in range(2):
                j = 2 * p + b
                compute(j, b, _sb)

                @pl.when(p < CPS // 2 - 1)
                def _():
                    issue_gather(_sb, j + 2, b)
                    issue_ee(_S, j + 2, b)
            return carry

        lax.fori_loop(p_lo, CPS // 2, pairf, 0)

        if S + 1 < NSUPER:
            drain(src2d.at[pl.ds(0, CPS)], SI[nb], IS[nb])
            drain(src2d.at[pl.ds(0, CPS)], DI[nb], IS[nb])
            for b in range(2):
                issue_gather(nb, b, b)
                issue_ee(S + 1, b, b)

    plsc.subcore_barrier()

    def mblk(g, carry):
        r0 = s * NPT + g * C
        pltpu.sync_copy(acc.at[pl.ds(r0, C)], rows0)

        @plsc.parallel_loop(0, C, unroll=2)
        def _(r):
            for k in range(4):
                sl = pl.ds(k * 16, 16)
                q = rows0[r, sl]
                p = rows0[r, pl.ds(HH + k * 16, 16)]
                ee0[r, sl] = q / (p + 1e-16)

        pltpu.sync_copy(ee0, m_out.at[pl.ds(c * NP + r0, C)])
        return carry

    lax.fori_loop(0, NPT // C, mblk, 0)


@functools.partial(
    pl.kernel,
    mesh=_MESH,
    out_type=jax.ShapeDtypeStruct((NP, H), jnp.float32),
    compiler_params=pltpu.CompilerParams(use_tc_tiling_on_sc=False),
    scratch_types=[
        pltpu.VMEM((GC,), jnp.int32),
        pltpu.VMEM((GR, H), jnp.float32),
        pltpu.SemaphoreType.DMA,
    ],
)
def _sc_gather(h0src, nidx, h0s, g_idx, gbuf, sem):
    c = lax.axis_index("c")
    s = lax.axis_index("s")
    rbase = (c * NS + s) * GR
    for g in range(GR // GC):
        pltpu.sync_copy(nidx.at[pl.ds(rbase + g * GC, GC)], g_idx)
        pltpu.async_copy(h0src.at[g_idx], gbuf.at[pl.ds(g * GC, GC)], sem).wait()
    pltpu.sync_copy(gbuf, h0s.at[pl.ds(rbase, GR)])


def _tc_in_proj_half_body(a_ref, w_ref, b_ref, o_ref):
    y = jnp.dot(a_ref[...], w_ref[...],
                preferred_element_type=jnp.float32) + b_ref[...]
    o_ref[0] = y[:, :HH]
    o_ref[1] = y[:, HH:]


def _in_proj_half(a, w, b, br):
    n = a.shape[0]
    return pl.pallas_call(
        _tc_in_proj_half_body,
        grid=(n // br,),
        in_specs=[
            pl.BlockSpec((br, 8), lambda i: (i, 0)),
            pl.BlockSpec((8, H), lambda i: (0, 0)),
            pl.BlockSpec((1, H), lambda i: (0, 0)),
        ],
        out_specs=pl.BlockSpec((2, br, HH), lambda i: (0, i, 0)),
        out_shape=jax.ShapeDtypeStruct((2, n, HH), jnp.float32),
    )(a, w, b.reshape(1, H))


def _tc_in_proj_full_body(a_ref, w_ref, b_ref, o_ref):
    o_ref[...] = jnp.dot(a_ref[...], w_ref[...],
                         preferred_element_type=jnp.float32) + b_ref[...]


def _in_proj_full(a, w, b, br):
    n = a.shape[0]
    return pl.pallas_call(
        _tc_in_proj_full_body,
        grid=(n // br,),
        in_specs=[
            pl.BlockSpec((br, 8), lambda i: (i, 0)),
            pl.BlockSpec((8, H), lambda i: (0, 0)),
            pl.BlockSpec((1, H), lambda i: (0, 0)),
        ],
        out_specs=pl.BlockSpec((br, H), lambda i: (i, 0)),
        out_shape=jax.ShapeDtypeStruct((n, H), jnp.float32),
    )(a, w, b.reshape(1, H))


def _ln_relu(hn, g, b):
    mu = jnp.mean(hn, axis=-1, keepdims=True)
    var = jnp.mean((hn - mu) ** 2, axis=-1, keepdims=True)
    return jnp.maximum(g * (hn - mu) / jnp.sqrt(var + 1e-5) + b, 0.0)


def _tc_layer_res_body(m_ref, h2_ref, hp_ref, w_ref, b_ref, g_ref, bb_ref,
                       hn_ref, h2n_ref):
    m = jnp.concatenate([m_ref[0], m_ref[1]], axis=-1)
    u = jnp.dot(h2_ref[...] + m, w_ref[...],
                preferred_element_type=jnp.float32) + b_ref[...]
    hn = u + hp_ref[...]
    hn_ref[...] = hn
    h2n_ref[...] = _ln_relu(hn, g_ref[...], bb_ref[...])


def _tc_layer0_body(m_ref, h2_ref, w_ref, b_ref, g_ref, bb_ref,
                    hn_ref, h2n_ref):
    m = jnp.concatenate([m_ref[0], m_ref[1]], axis=-1)
    hn = jnp.dot(h2_ref[...] + m, w_ref[...],
                 preferred_element_type=jnp.float32) + b_ref[...]
    hn_ref[...] = hn
    h2n_ref[...] = _ln_relu(hn, g_ref[...], bb_ref[...])


def _tc_layer(m2, h2, hp, w, b, g, bb, br=1024, residual=True):
    half = pl.BlockSpec((2, br, HH), lambda i: (0, i, 0))
    full = pl.BlockSpec((br, H), lambda i: (i, 0))
    wspec = pl.BlockSpec((H, H), lambda i: (0, 0))
    vspec = pl.BlockSpec((1, H), lambda i: (0, 0))
    body = _tc_layer_res_body if residual else _tc_layer0_body
    in_specs = [half, full] + ([full] if residual else []) + [wspec, vspec, vspec, vspec]
    args = [m2, h2] + ([hp] if residual else []) + \
        [w, b.reshape(1, H), g.reshape(1, H), bb.reshape(1, H)]
    return pl.pallas_call(
        body,
        grid=(NP // br,),
        in_specs=in_specs,
        out_specs=[full, full],
        out_shape=[jax.ShapeDtypeStruct((NP, H), jnp.float32),
                   jax.ShapeDtypeStruct((NP, H), jnp.float32)],
    )(*args)


def _tc_final_body(m_ref, h2_ref, hp_ref, w_ref, b_ref, g_ref, bb_ref,
                   wp_ref, bp_ref, o_ref):
    m = jnp.concatenate([m_ref[0], m_ref[1]], axis=-1)
    u = jnp.dot(h2_ref[...] + m, w_ref[...],
                preferred_element_type=jnp.float32) + b_ref[...]
    hn = u + hp_ref[...]
    y = _ln_relu(hn, g_ref[...], bb_ref[...])
    o_ref[...] = jnp.dot(y, wp_ref[...],
                         preferred_element_type=jnp.float32) + bp_ref[...]


def _tc_final(m2, h2, hp, w, b, g, bb, wp, bp, br=1024):
    half = pl.BlockSpec((2, br, HH), lambda i: (0, i, 0))
    full = pl.BlockSpec((br, H), lambda i: (i, 0))
    wspec = pl.BlockSpec((H, H), lambda i: (0, 0))
    vspec = pl.BlockSpec((1, H), lambda i: (0, 0))
    return pl.pallas_call(
        _tc_final_body,
        grid=(NP // br,),
        in_specs=[half, full, full, wspec, vspec, vspec, vspec,
                  pl.BlockSpec((H, NUM_TASKS), lambda i: (0, 0)),
                  pl.BlockSpec((1, NUM_TASKS), lambda i: (0, 0))],
        out_specs=pl.BlockSpec((br, NUM_TASKS), lambda i: (i, 0)),
        out_shape=jax.ShapeDtypeStruct((NP, NUM_TASKS), jnp.float32),
    )(m2, h2, hp, w, b.reshape(1, H), g.reshape(1, H), bb.reshape(1, H),
      wp, bp.reshape(1, NUM_TASKS))


def kernel(x, node_index, edge_index, edge_attr, node_features, W_nf, b_nf,
           W_edge, b_edge, Wg, bg, ln_g, ln_b, t, W_pred, b_pred):
    src = edge_index[0].astype(jnp.int32)
    dst = edge_index[1].astype(jnp.int32)
    nodei = node_index.astype(jnp.int32)

    srcp = jnp.zeros((E_PAD,), jnp.int32).at[:E].set(src)
    dstp = jnp.full((E_PAD,), DEAD_ROW, jnp.int32).at[:E].set(dst)
    eap = jnp.zeros((E_PAD, 8), jnp.float32).at[:E].set(edge_attr)
    nip = jnp.zeros((NP,), jnp.int32).at[:N].set(nodei)
    tvs = jnp.broadcast_to(t.reshape(NUM_LAYERS, 1), (NUM_LAYERS, L))
    nfp = jnp.zeros((NP, 8), jnp.float32).at[:N].set(node_features)

    h0f = _in_proj_full(nfp, W_nf, b_nf, br=1024)
    eeh = _in_proj_half(eap, W_edge, b_edge, br=2048)
    eep = eeh.reshape(2 * E_PAD, HH)

    src2d = srcp.reshape(E_PAD // C, C)
    dst2d = dstp.reshape(E_PAD // C, C)

    h2 = _sc_gather(h0f, nip)
    h = None
    for l in range(NUM_LAYERS):
        m_f = _sc_edge(h2, src2d, dst2d, eep, tvs[l])
        m2 = m_f.reshape(2, NP, HH)
        if l == 0:
            h, h2 = _tc_layer(m2, h2, None, Wg[0], bg[0], ln_g[0], ln_b[0],
                              residual=False)
        elif l < NUM_LAYERS - 1:
            h, h2 = _tc_layer(m2, h2, h, Wg[l], bg[l], ln_g[l], ln_b[l])
        else:
            out = _tc_final(m2, h2, h, Wg[l], bg[l],
                            ln_g[l], ln_b[l], W_pred, b_pred)
            return out[:N]


# speedup vs baseline: 4.6571x; 1.0353x over previous
import functools

import jax
import jax.numpy as jnp
from jax import lax
from jax.experimental import pallas as pl
from jax.experimental.pallas import tpu as pltpu
from jax.experimental.pallas import tpu_sc as plsc

N = 10000
NP = 10240
E = 320000
H = 128
HH = 64
NUM_LAYERS = 7
NUM_TASKS = 112
MSG_EPS = 1e-7

NC, NS, L = 2, 16, 16
C = 32
CPS = 64
SUP = C * CPS
NSUPER = 10
EPT = SUP * NSUPER
E_PAD = NS * EPT
ACC_ROWS = NP
DEAD_ROW = N + 100
NPT = NP // NS
GR = NP // (NC * NS)
GC = 64

_MESH = plsc.VectorSubcoreMesh(core_axis_name="c", subcore_axis_name="s")


@functools.partial(
    pl.kernel,
    mesh=_MESH,
    out_type=jax.ShapeDtypeStruct((2 * NP, HH), jnp.float32),
    compiler_params=pltpu.CompilerParams(use_tc_tiling_on_sc=False),
    scratch_types=[
        pltpu.VMEM_SHARED((ACC_ROWS, H), jnp.float32),
        pltpu.VMEM((CPS, C), jnp.int32),
        pltpu.VMEM((CPS, C), jnp.int32),
        pltpu.VMEM((CPS, C), jnp.int32),
        pltpu.VMEM((CPS, C), jnp.int32),
        pltpu.VMEM((C, H), jnp.float32),
        pltpu.VMEM((C, H), jnp.float32),
        pltpu.VMEM((C, HH), jnp.float32),
        pltpu.VMEM((C, HH), jnp.float32),
        pltpu.VMEM((C, H), jnp.float32),
        pltpu.VMEM((C, H), jnp.float32),
        pltpu.VMEM((L,), jnp.float32),
    ] + [pltpu.SemaphoreType.DMA] * 8,
)
def _sc_edge(h2s, src2d, dst2d, eep, tvec, m_out, acc,
             si0, si1, di0, di1, rows0, rows1, ee0, ee1, pq0, pq1, tvm,
             is0, is1, gs0, gs1, es0, es1, ss0, ss1):
    c = lax.axis_index("c")
    s = lax.axis_index("s")
    co = c * HH
    SI, DI = (si0, si1), (di0, di1)
    RO, EB, PQ = (rows0, rows1), (ee0, ee1), (pq0, pq1)
    IS, GS, ES, SS = (is0, is1), (gs0, gs1), (es0, es1), (ss0, ss1)

    pltpu.sync_copy(tvec, tvm)
    tv = tvm[...]

    def zrow(r, carry):
        for k in range(8):
            pq0[r, pl.ds(k * 16, 16)] = jnp.zeros((16,), jnp.float32)
        return carry

    lax.fori_loop(0, C, zrow, 0)

    def zcp(g, carry):
        pltpu.sync_copy(pq0, acc.at[pl.ds(s * NPT + g * C, C)])
        return carry

    lax.fori_loop(0, NPT // C, zcp, 0)
    plsc.subcore_barrier()

    irow = s * (EPT // C)
    ebase = s * EPT

    def issue_gather(sb, j, b):
        pltpu.async_copy(h2s.at[SI[sb].at[j]], RO[b], GS[b])

    def issue_ee(S, j, b):
        off = c * E_PAD + ebase + S * SUP + j * C
        pltpu.async_copy(eep.at[pl.ds(off, C)], EB[b], ES[b])

    def drain(dummy, buf, sem):
        pltpu.make_async_copy(dummy, buf, sem).wait()

    def compute(j, b, sb, scatter_drain=True):
        drain(h2s.at[pl.ds(0, C)], RO[b], GS[b])
        drain(eep.at[pl.ds(0, C)], EB[b], ES[b])
        if scatter_drain:
            drain(h2s.at[pl.ds(0, C)], PQ[b], SS[b])

        @plsc.parallel_loop(0, C, unroll=2)
        def _(r):
            for k in range(4):
                sl = pl.ds(k * 16, 16)
                msg = (jnp.maximum(RO[b][r, pl.ds(co + k * 16, 16)]
                                   + EB[b][r, sl], 0.0) + MSG_EPS)
                p = jnp.exp(msg * tv)
                PQ[b][r, pl.ds(HH + k * 16, 16)] = p
                PQ[b][r, sl] = msg * p

        pltpu.async_copy(PQ[b], acc.at[DI[sb].at[j]], SS[b], add=True)

    pltpu.sync_copy(src2d.at[pl.ds(irow, CPS)], si0)
    pltpu.sync_copy(dst2d.at[pl.ds(irow, CPS)], di0)
    for b in range(2):
        issue_gather(0, b, b)
        issue_ee(0, b, b)

    for S in range(NSUPER):
        sb = S % 2
        nb = 1 - sb
        if S + 1 < NSUPER:
            pltpu.async_copy(src2d.at[pl.ds(irow + (S + 1) * CPS, CPS)],
                             SI[nb], IS[nb])
            pltpu.async_copy(dst2d.at[pl.ds(irow + (S + 1) * CPS, CPS)],
                             DI[nb], IS[nb])
        if S == 0:
            for b in range(2):
                compute(b, b, sb, scatter_drain=False)
                issue_gather(sb, b + 2, b)
                issue_ee(S, b + 2, b)
            p_lo = 1
        else:
            p_lo = 0

        def pairf(p, carry, _S=S, _sb=sb):
            for b in range(2):
                j = 2 * p + b
                compute(j, b, _sb)

                @pl.when(p < CPS // 2 - 1)
                def _():
                    issue_gather(_sb, j + 2, b)
                    issue_ee(_S, j + 2, b)
            return carry

        lax.fori_loop(p_lo, CPS // 2, pairf, 0)

        if S + 1 < NSUPER:
            drain(src2d.at[pl.ds(0, CPS)], SI[nb], IS[nb])
            drain(src2d.at[pl.ds(0, CPS)], DI[nb], IS[nb])
            for b in range(2):
                issue_gather(nb, b, b)
                issue_ee(S + 1, b, b)

    for b in range(2):
        drain(h2s.at[pl.ds(0, C)], PQ[b], SS[b])
    plsc.subcore_barrier()

    def mblk(g, carry):
        r0 = s * NPT + g * C
        pltpu.sync_copy(acc.at[pl.ds(r0, C)], rows0)

        @plsc.parallel_loop(0, C, unroll=2)
        def _(r):
            for k in range(4):
                sl = pl.ds(k * 16, 16)
                q = rows0[r, sl]
                p = rows0[r, pl.ds(HH + k * 16, 16)]
                ee0[r, sl] = q / (p + 1e-16)

        pltpu.sync_copy(ee0, m_out.at[pl.ds(c * NP + r0, C)])
        return carry

    lax.fori_loop(0, NPT // C, mblk, 0)


@functools.partial(
    pl.kernel,
    mesh=_MESH,
    out_type=jax.ShapeDtypeStruct((NP, H), jnp.float32),
    compiler_params=pltpu.CompilerParams(use_tc_tiling_on_sc=False),
    scratch_types=[
        pltpu.VMEM((GC,), jnp.int32),
        pltpu.VMEM((GR, H), jnp.float32),
        pltpu.SemaphoreType.DMA,
    ],
)
def _sc_gather(h0src, nidx, h0s, g_idx, gbuf, sem):
    c = lax.axis_index("c")
    s = lax.axis_index("s")
    rbase = (c * NS + s) * GR
    for g in range(GR // GC):
        pltpu.sync_copy(nidx.at[pl.ds(rbase + g * GC, GC)], g_idx)
        pltpu.async_copy(h0src.at[g_idx], gbuf.at[pl.ds(g * GC, GC)], sem).wait()
    pltpu.sync_copy(gbuf, h0s.at[pl.ds(rbase, GR)])


def _tc_in_proj_half_body(a_ref, w_ref, b_ref, o_ref):
    y = jnp.dot(a_ref[...], w_ref[...],
                preferred_element_type=jnp.float32) + b_ref[...]
    o_ref[0] = y[:, :HH]
    o_ref[1] = y[:, HH:]


def _in_proj_half(a, w, b, br):
    n = a.shape[0]
    return pl.pallas_call(
        _tc_in_proj_half_body,
        grid=(n // br,),
        in_specs=[
            pl.BlockSpec((br, 8), lambda i: (i, 0)),
            pl.BlockSpec((8, H), lambda i: (0, 0)),
            pl.BlockSpec((1, H), lambda i: (0, 0)),
        ],
        out_specs=pl.BlockSpec((2, br, HH), lambda i: (0, i, 0)),
        out_shape=jax.ShapeDtypeStruct((2, n, HH), jnp.float32),
    )(a, w, b.reshape(1, H))


def _tc_in_proj_full_body(a_ref, w_ref, b_ref, o_ref):
    o_ref[...] = jnp.dot(a_ref[...], w_ref[...],
                         preferred_element_type=jnp.float32) + b_ref[...]


def _in_proj_full(a, w, b, br):
    n = a.shape[0]
    return pl.pallas_call(
        _tc_in_proj_full_body,
        grid=(n // br,),
        in_specs=[
            pl.BlockSpec((br, 8), lambda i: (i, 0)),
            pl.BlockSpec((8, H), lambda i: (0, 0)),
            pl.BlockSpec((1, H), lambda i: (0, 0)),
        ],
        out_specs=pl.BlockSpec((br, H), lambda i: (i, 0)),
        out_shape=jax.ShapeDtypeStruct((n, H), jnp.float32),
    )(a, w, b.reshape(1, H))


def _ln_relu(hn, g, b):
    mu = jnp.mean(hn, axis=-1, keepdims=True)
    var = jnp.mean((hn - mu) ** 2, axis=-1, keepdims=True)
    return jnp.maximum(g * (hn - mu) / jnp.sqrt(var + 1e-5) + b, 0.0)


def _tc_layer_res_body(m_ref, h2_ref, hp_ref, w_ref, b_ref, g_ref, bb_ref,
                       hn_ref, h2n_ref):
    m = jnp.concatenate([m_ref[0], m_ref[1]], axis=-1)
    u = jnp.dot(h2_ref[...] + m, w_ref[...],
                preferred_element_type=jnp.float32) + b_ref[...]
    hn = u + hp_ref[...]
    hn_ref[...] = hn
    h2n_ref[...] = _ln_relu(hn, g_ref[...], bb_ref[...])


def _tc_layer0_body(m_ref, h2_ref, w_ref, b_ref, g_ref, bb_ref,
                    hn_ref, h2n_ref):
    m = jnp.concatenate([m_ref[0], m_ref[1]], axis=-1)
    hn = jnp.dot(h2_ref[...] + m, w_ref[...],
                 preferred_element_type=jnp.float32) + b_ref[...]
    hn_ref[...] = hn
    h2n_ref[...] = _ln_relu(hn, g_ref[...], bb_ref[...])


def _tc_layer(m2, h2, hp, w, b, g, bb, br=1024, residual=True):
    half = pl.BlockSpec((2, br, HH), lambda i: (0, i, 0))
    full = pl.BlockSpec((br, H), lambda i: (i, 0))
    wspec = pl.BlockSpec((H, H), lambda i: (0, 0))
    vspec = pl.BlockSpec((1, H), lambda i: (0, 0))
    body = _tc_layer_res_body if residual else _tc_layer0_body
    in_specs = [half, full] + ([full] if residual else []) + [wspec, vspec, vspec, vspec]
    args = [m2, h2] + ([hp] if residual else []) + \
        [w, b.reshape(1, H), g.reshape(1, H), bb.reshape(1, H)]
    return pl.pallas_call(
        body,
        grid=(NP // br,),
        in_specs=in_specs,
        out_specs=[full, full],
        out_shape=[jax.ShapeDtypeStruct((NP, H), jnp.float32),
                   jax.ShapeDtypeStruct((NP, H), jnp.float32)],
    )(*args)


def _tc_final_body(m_ref, h2_ref, hp_ref, w_ref, b_ref, g_ref, bb_ref,
                   wp_ref, bp_ref, o_ref):
    m = jnp.concatenate([m_ref[0], m_ref[1]], axis=-1)
    u = jnp.dot(h2_ref[...] + m, w_ref[...],
                preferred_element_type=jnp.float32) + b_ref[...]
    hn = u + hp_ref[...]
    y = _ln_relu(hn, g_ref[...], bb_ref[...])
    o_ref[...] = jnp.dot(y, wp_ref[...],
                         preferred_element_type=jnp.float32) + bp_ref[...]


def _tc_final(m2, h2, hp, w, b, g, bb, wp, bp, br=1024):
    half = pl.BlockSpec((2, br, HH), lambda i: (0, i, 0))
    full = pl.BlockSpec((br, H), lambda i: (i, 0))
    wspec = pl.BlockSpec((H, H), lambda i: (0, 0))
    vspec = pl.BlockSpec((1, H), lambda i: (0, 0))
    return pl.pallas_call(
        _tc_final_body,
        grid=(NP // br,),
        in_specs=[half, full, full, wspec, vspec, vspec, vspec,
                  pl.BlockSpec((H, NUM_TASKS), lambda i: (0, 0)),
                  pl.BlockSpec((1, NUM_TASKS), lambda i: (0, 0))],
        out_specs=pl.BlockSpec((br, NUM_TASKS), lambda i: (i, 0)),
        out_shape=jax.ShapeDtypeStruct((NP, NUM_TASKS), jnp.float32),
    )(m2, h2, hp, w, b.reshape(1, H), g.reshape(1, H), bb.reshape(1, H),
      wp, bp.reshape(1, NUM_TASKS))


def kernel(x, node_index, edge_index, edge_attr, node_features, W_nf, b_nf,
           W_edge, b_edge, Wg, bg, ln_g, ln_b, t, W_pred, b_pred):
    src = edge_index[0].astype(jnp.int32)
    dst = edge_index[1].astype(jnp.int32)
    nodei = node_index.astype(jnp.int32)

    srcp = jnp.zeros((E_PAD,), jnp.int32).at[:E].set(src)
    dstp = jnp.full((E_PAD,), DEAD_ROW, jnp.int32).at[:E].set(dst)
    eap = jnp.zeros((E_PAD, 8), jnp.float32).at[:E].set(edge_attr)
    nip = jnp.zeros((NP,), jnp.int32).at[:N].set(nodei)
    tvs = jnp.broadcast_to(t.reshape(NUM_LAYERS, 1), (NUM_LAYERS, L))
    nfp = jnp.zeros((NP, 8), jnp.float32).at[:N].set(node_features)

    h0f = _in_proj_full(nfp, W_nf, b_nf, br=1024)
    eeh = _in_proj_half(eap, W_edge, b_edge, br=2048)
    eep = eeh.reshape(2 * E_PAD, HH)

    src2d = srcp.reshape(E_PAD // C, C)
    dst2d = dstp.reshape(E_PAD // C, C)

    h2 = _sc_gather(h0f, nip)
    h = None
    for l in range(NUM_LAYERS):
        m_f = _sc_edge(h2, src2d, dst2d, eep, tvs[l])
        m2 = m_f.reshape(2, NP, HH)
        if l == 0:
            h, h2 = _tc_layer(m2, h2, None, Wg[0], bg[0], ln_g[0], ln_b[0],
                              residual=False)
        elif l < NUM_LAYERS - 1:
            h, h2 = _tc_layer(m2, h2, h, Wg[l], bg[l], ln_g[l], ln_b[l])
        else:
            out = _tc_final(m2, h2, h, Wg[l], bg[l],
                            ln_g[l], ln_b[l], W_pred, b_pred)
            return out[:N]


# speedup vs baseline: 8.7389x; 1.8765x over previous
import functools

import jax
import jax.numpy as jnp
from jax import lax
from jax.experimental import pallas as pl
from jax.experimental.pallas import tpu as pltpu
from jax.experimental.pallas import tpu_sc as plsc

N = 10000
NP = 10240
E = 320000
H = 128
HH = 64
NUM_LAYERS = 7
NUM_TASKS = 112
MSG_EPS = 1e-7

NC, NS, L = 2, 16, 16
C = 64
CPS = 32
SUP = C * CPS
NSUPER = 10
EPT = SUP * NSUPER
E_PAD = NS * EPT
ACC_ROWS = NP
DEAD_ROW = N + 100
NPT = NP // NS

_MESH = plsc.VectorSubcoreMesh(core_axis_name="c", subcore_axis_name="s")


@functools.partial(
    pl.kernel,
    mesh=_MESH,
    out_type=jax.ShapeDtypeStruct((2 * NP, HH), jnp.float32),
    compiler_params=pltpu.CompilerParams(use_tc_tiling_on_sc=False),
    scratch_types=[
        pltpu.VMEM_SHARED((ACC_ROWS, H), jnp.float32),
        pltpu.VMEM((CPS, C), jnp.int32),
        pltpu.VMEM((CPS, C), jnp.int32),
        pltpu.VMEM((CPS, C), jnp.int32),
        pltpu.VMEM((CPS, C), jnp.int32),
        pltpu.VMEM((C, HH), jnp.float32),
        pltpu.VMEM((C, HH), jnp.float32),
        pltpu.VMEM((C, HH), jnp.float32),
        pltpu.VMEM((C, HH), jnp.float32),
        pltpu.VMEM((C, H), jnp.float32),
        pltpu.VMEM((C, H), jnp.float32),
        pltpu.VMEM((L,), jnp.float32),
    ] + [pltpu.SemaphoreType.DMA] * 8,
)
def _sc_edge(h2s, src2d, dst2d, eep, tvec, m_out, acc,
             si0, si1, di0, di1, rows0, rows1, ee0, ee1, pq0, pq1, tvm,
             is0, is1, gs0, gs1, es0, es1, ss0, ss1):
    c = lax.axis_index("c")
    s = lax.axis_index("s")
    SI, DI = (si0, si1), (di0, di1)
    RO, EB, PQ = (rows0, rows1), (ee0, ee1), (pq0, pq1)
    IS, GS, ES, SS = (is0, is1), (gs0, gs1), (es0, es1), (ss0, ss1)

    pltpu.sync_copy(tvec, tvm)
    tv = tvm[...]

    def zrow(r, carry):
        for k in range(8):
            pq0[r, pl.ds(k * 16, 16)] = jnp.zeros((16,), jnp.float32)
        return carry

    lax.fori_loop(0, C, zrow, 0)

    def zcp(g, carry):
        pltpu.sync_copy(pq0, acc.at[pl.ds(s * NPT + g * C, C)])
        return carry

    lax.fori_loop(0, NPT // C, zcp, 0)
    plsc.subcore_barrier()

    irow = c * (E_PAD // C) + s * (EPT // C)
    drow = s * (EPT // C)
    ebase = s * EPT

    def issue_gather(sb, j, b):
        pltpu.async_copy(h2s.at[SI[sb].at[j]], RO[b], GS[b])

    def issue_ee(S, j, b):
        off = c * E_PAD + ebase + S * SUP + j * C
        pltpu.async_copy(eep.at[pl.ds(off, C)], EB[b], ES[b])

    def drain(dummy, buf, sem):
        pltpu.make_async_copy(dummy, buf, sem).wait()

    def compute(j, b, sb, scatter_drain=True):
        drain(eep.at[pl.ds(0, C)], RO[b], GS[b])
        drain(eep.at[pl.ds(0, C)], EB[b], ES[b])
        if scatter_drain:
            drain(h2s.at[pl.ds(0, C)], PQ[b], SS[b])

        @plsc.parallel_loop(0, C, unroll=2)
        def _(r):
            for k in range(4):
                sl = pl.ds(k * 16, 16)
                msg = (jnp.maximum(RO[b][r, sl] + EB[b][r, sl], 0.0)
                       + MSG_EPS)
                p = jnp.exp(msg * tv)
                PQ[b][r, pl.ds(HH + k * 16, 16)] = p
                PQ[b][r, sl] = msg * p

        pltpu.async_copy(PQ[b], acc.at[DI[sb].at[j]], SS[b], add=True)

    pltpu.sync_copy(src2d.at[pl.ds(irow, CPS)], si0)
    pltpu.sync_copy(dst2d.at[pl.ds(drow, CPS)], di0)
    for b in range(2):
        issue_gather(0, b, b)
        issue_ee(0, b, b)

    for S in range(NSUPER):
        sb = S % 2
        nb = 1 - sb
        if S + 1 < NSUPER:
            pltpu.async_copy(src2d.at[pl.ds(irow + (S + 1) * CPS, CPS)],
                             SI[nb], IS[nb])
            pltpu.async_copy(dst2d.at[pl.ds(drow + (S + 1) * CPS, CPS)],
                             DI[nb], IS[nb])
        if S == 0:
            for b in range(2):
                compute(b, b, sb, scatter_drain=False)
                issue_gather(sb, b + 2, b)
                issue_ee(S, b + 2, b)
            p_lo = 1
        else:
            p_lo = 0

        def pairf(p, carry, _S=S, _sb=sb):
            for b in range(2):
                j = 2 * p + b
                compute(j, b, _sb)

                @pl.when(p < CPS // 2 - 1)
                def _():
                    issue_gather(_sb, j + 2, b)
                    issue_ee(_S, j + 2, b)
            return carry

        lax.fori_loop(p_lo, CPS // 2, pairf, 0)

        if S + 1 < NSUPER:
            drain(src2d.at[pl.ds(0, CPS)], SI[nb], IS[nb])
            drain(src2d.at[pl.ds(0, CPS)], DI[nb], IS[nb])
            for b in range(2):
                issue_gather(nb, b, b)
                issue_ee(S + 1, b, b)

    for b in range(2):
        drain(h2s.at[pl.ds(0, C)], PQ[b], SS[b])
    plsc.subcore_barrier()

    def mblk(g, carry):
        r0 = s * NPT + g * C
        pltpu.sync_copy(acc.at[pl.ds(r0, C)], pq0)

        @plsc.parallel_loop(0, C, unroll=2)
        def _(r):
            for k in range(4):
                sl = pl.ds(k * 16, 16)
                q = pq0[r, sl]
                p = pq0[r, pl.ds(HH + k * 16, 16)]
                ee0[r, sl] = q / (p + 1e-16)

        pltpu.sync_copy(ee0, m_out.at[pl.ds(c * NP + r0, C)])
        return carry

    lax.fori_loop(0, NPT // C, mblk, 0)


@functools.partial(
    pl.kernel,
    mesh=_MESH,
    out_type=jax.ShapeDtypeStruct((2 * NP, HH), jnp.float32),
    compiler_params=pltpu.CompilerParams(use_tc_tiling_on_sc=False),
    scratch_types=[
        pltpu.VMEM((C,), jnp.int32),
        pltpu.VMEM((C, HH), jnp.float32),
        pltpu.SemaphoreType.DMA,
    ],
)
def _sc_gather(h0src, nidx2, h0s, g_idx, gbuf, sem):
    c = lax.axis_index("c")
    s = lax.axis_index("s")
    rbase = c * NP + s * NPT

    def blk(g, carry):
        r0 = rbase + g * C
        pltpu.sync_copy(nidx2.at[pl.ds(r0, C)], g_idx)
        pltpu.async_copy(h0src.at[g_idx], gbuf, sem).wait()
        pltpu.sync_copy(gbuf, h0s.at[pl.ds(r0, C)])
        return carry

    lax.fori_loop(0, NPT // C, blk, 0)


def _tc_in_proj_body(a_ref, w_ref, b_ref, o_ref):
    y = jnp.dot(a_ref[...], w_ref[...],
                preferred_element_type=jnp.float32) + b_ref[...]
    o_ref[0] = y[:, :HH]
    o_ref[1] = y[:, HH:]


def _in_proj(a, w, b, br):
    n = a.shape[0]
    return pl.pallas_call(
        _tc_in_proj_body,
        grid=(n // br,),
        in_specs=[
            pl.BlockSpec((br, 8), lambda i: (i, 0)),
            pl.BlockSpec((8, H), lambda i: (0, 0)),
            pl.BlockSpec((1, H), lambda i: (0, 0)),
        ],
        out_specs=pl.BlockSpec((2, br, HH), lambda i: (0, i, 0)),
        out_shape=jax.ShapeDtypeStruct((2, n, HH), jnp.float32),
    )(a, w, b.reshape(1, H))


def _ln_relu(hn, g, b):
    mu = jnp.mean(hn, axis=-1, keepdims=True)
    var = jnp.mean((hn - mu) ** 2, axis=-1, keepdims=True)
    return jnp.maximum(g * (hn - mu) / jnp.sqrt(var + 1e-5) + b, 0.0)


def _tc_layer_res_body(m_ref, h2_ref, hp_ref, w_ref, b_ref, g_ref, bb_ref,
                       hn_ref, h2n_ref):
    m = jnp.concatenate([m_ref[0], m_ref[1]], axis=-1)
    h2 = jnp.concatenate([h2_ref[0], h2_ref[1]], axis=-1)
    u = jnp.dot(h2 + m, w_ref[...],
                preferred_element_type=jnp.float32) + b_ref[...]
    hn = u + hp_ref[...]
    hn_ref[...] = hn
    y = _ln_relu(hn, g_ref[...], bb_ref[...])
    h2n_ref[0] = y[:, :HH]
    h2n_ref[1] = y[:, HH:]


def _tc_layer0_body(m_ref, h2_ref, w_ref, b_ref, g_ref, bb_ref,
                    hn_ref, h2n_ref):
    m = jnp.concatenate([m_ref[0], m_ref[1]], axis=-1)
    h2 = jnp.concatenate([h2_ref[0], h2_ref[1]], axis=-1)
    hn = jnp.dot(h2 + m, w_ref[...],
                 preferred_element_type=jnp.float32) + b_ref[...]
    hn_ref[...] = hn
    y = _ln_relu(hn, g_ref[...], bb_ref[...])
    h2n_ref[0] = y[:, :HH]
    h2n_ref[1] = y[:, HH:]


def _tc_layer(m2, h2h, hp, w, b, g, bb, br=1024, residual=True):
    half = pl.BlockSpec((2, br, HH), lambda i: (0, i, 0))
    full = pl.BlockSpec((br, H), lambda i: (i, 0))
    wspec = pl.BlockSpec((H, H), lambda i: (0, 0))
    vspec = pl.BlockSpec((1, H), lambda i: (0, 0))
    body = _tc_layer_res_body if residual else _tc_layer0_body
    in_specs = [half, half] + ([full] if residual else []) + \
        [wspec, vspec, vspec, vspec]
    args = [m2, h2h] + ([hp] if residual else []) + \
        [w, b.reshape(1, H), g.reshape(1, H), bb.reshape(1, H)]
    return pl.pallas_call(
        body,
        grid=(NP // br,),
        in_specs=in_specs,
        out_specs=[full, half],
        out_shape=[jax.ShapeDtypeStruct((NP, H), jnp.float32),
                   jax.ShapeDtypeStruct((2, NP, HH), jnp.float32)],
    )(*args)


def _tc_final_body(m_ref, h2_ref, hp_ref, w_ref, b_ref, g_ref, bb_ref,
                   wp_ref, bp_ref, o_ref):
    m = jnp.concatenate([m_ref[0], m_ref[1]], axis=-1)
    h2 = jnp.concatenate([h2_ref[0], h2_ref[1]], axis=-1)
    u = jnp.dot(h2 + m, w_ref[...],
                preferred_element_type=jnp.float32) + b_ref[...]
    hn = u + hp_ref[...]
    y = _ln_relu(hn, g_ref[...], bb_ref[...])
    o_ref[...] = jnp.dot(y, wp_ref[...],
                         preferred_element_type=jnp.float32) + bp_ref[...]


def _tc_final(m2, h2h, hp, w, b, g, bb, wp, bp, br=1024):
    half = pl.BlockSpec((2, br, HH), lambda i: (0, i, 0))
    full = pl.BlockSpec((br, H), lambda i: (i, 0))
    wspec = pl.BlockSpec((H, H), lambda i: (0, 0))
    vspec = pl.BlockSpec((1, H), lambda i: (0, 0))
    return pl.pallas_call(
        _tc_final_body,
        grid=(NP // br,),
        in_specs=[half, half, full, wspec, vspec, vspec, vspec,
                  pl.BlockSpec((H, NUM_TASKS), lambda i: (0, 0)),
                  pl.BlockSpec((1, NUM_TASKS), lambda i: (0, 0))],
        out_specs=pl.BlockSpec((br, NUM_TASKS), lambda i: (i, 0)),
        out_shape=jax.ShapeDtypeStruct((NP, NUM_TASKS), jnp.float32),
    )(m2, h2h, hp, w, b.reshape(1, H), g.reshape(1, H), bb.reshape(1, H),
      wp, bp.reshape(1, NUM_TASKS))


def kernel(x, node_index, edge_index, edge_attr, node_features, W_nf, b_nf,
           W_edge, b_edge, Wg, bg, ln_g, ln_b, t, W_pred, b_pred):
    src = edge_index[0].astype(jnp.int32)
    dst = edge_index[1].astype(jnp.int32)
    nodei = node_index.astype(jnp.int32)

    srcp = jnp.zeros((E_PAD,), jnp.int32).at[:E].set(src)
    src2d = jnp.concatenate([srcp, srcp + NP]).reshape(2 * E_PAD // C, C)
    dstp = jnp.full((E_PAD,), DEAD_ROW, jnp.int32).at[:E].set(dst)
    dst2d = dstp.reshape(E_PAD // C, C)
    eap = jnp.zeros((E_PAD, 8), jnp.float32).at[:E].set(edge_attr)
    nip = jnp.zeros((NP,), jnp.int32).at[:N].set(nodei)
    nidx2 = jnp.concatenate([nip, nip + NP])
    tvs = jnp.broadcast_to(t.reshape(NUM_LAYERS, 1), (NUM_LAYERS, L))
    nfp = jnp.zeros((NP, 8), jnp.float32).at[:N].set(node_features)

    h0h = _in_proj(nfp, W_nf, b_nf, br=1024)
    eeh = _in_proj(eap, W_edge, b_edge, br=2048)
    eep = eeh.reshape(2 * E_PAD, HH)

    h2f = _sc_gather(h0h.reshape(2 * NP, HH), nidx2)
    h2h = h2f.reshape(2, NP, HH)
    h = None
    for l in range(NUM_LAYERS):
        m_f = _sc_edge(h2f, src2d, dst2d, eep, tvs[l])
        m2 = m_f.reshape(2, NP, HH)
        if l == 0:
            h, h2h = _tc_layer(m2, h2h, None, Wg[0], bg[0], ln_g[0], ln_b[0],
                               residual=False)
        elif l < NUM_LAYERS - 1:
            h, h2h = _tc_layer(m2, h2h, h, Wg[l], bg[l], ln_g[l], ln_b[l])
        else:
            out = _tc_final(m2, h2h, h, Wg[l], bg[l],
                            ln_g[l], ln_b[l], W_pred, b_pred)
            return out[:N]
        h2f = h2h.reshape(2 * NP, HH)
